# baseline (speedup 1.0000x reference)
import functools
import jax
import jax.numpy as jnp
from jax import lax
from jax.experimental import pallas as pl
from jax.experimental.pallas import tpu as pltpu
from jax.experimental.pallas import tpu_sc as plsc

N = 10000
D = 128
H2 = 64
E = 320000
NC, NS = 2, 16
NP = 10240
EPW = (2 * E) // (NC * NS)
CH = 40
NCHUNK = EPW // CH
RPW = NP // NS
NB = 4
SUP = 20
NSUP = NCHUNK // SUP


def _seg_body(with_counts, *refs):
    if with_counts:
        (feat, src_c, dst_c, zrows, ones_h, sums_out, cnt_out,
         src_idx, dst_idx, r0, r1, r2, r3, acc,
         g0, g1, g2, g3, s0, s1, s2, s3) = refs
    else:
        (feat, src_c, dst_c, zrows, sums_out,
         src_idx, dst_idx, r0, r1, r2, r3, acc,
         g0, g1, g2, g3, s0, s1, s2, s3) = refs
    rows = (r0, r1, r2, r3)
    gsem = (g0, g1, g2, g3)
    ssem = (s0, s1, s2, s3)
    c = lax.axis_index("c")
    s = lax.axis_index("s")
    wid = c * NS + s
    row0 = c * NP + s * RPW

    def g_fire(ci, j):
        pltpu.async_copy(feat.at[src_idx.at[ci, 0]], rows[j], gsem[j])

    def g_wait(ci, j):
        pltpu.make_async_copy(feat.at[src_idx.at[ci, 0]], rows[j],
                              gsem[j]).wait()

    def s_fire(ci, j):
        pltpu.async_copy(rows[j], acc.at[dst_idx.at[ci, 0]], ssem[j],
                         add=True)

    def s_wait(ci, j):
        pltpu.make_async_copy(rows[j], acc.at[dst_idx.at[ci, 0]],
                              ssem[j]).wait()

    def zero_acc():
        pltpu.sync_copy(zrows, acc.at[pl.ds(s * RPW, RPW)])
        plsc.subcore_barrier()

    def write_acc(out):
        pltpu.sync_copy(acc.at[pl.ds(s * RPW, RPW)], out.at[pl.ds(row0, RPW)])
        plsc.subcore_barrier()

    zero_acc()

    def sup_main(t, carry):
        pltpu.sync_copy(src_c.at[wid, t], src_idx)
        pltpu.sync_copy(dst_c.at[wid, t], dst_idx)
        g_fire(0, 0)
        g_fire(1, 1)
        g_wait(0, 0); s_fire(0, 0); g_fire(2, 2)
        g_wait(1, 1); s_fire(1, 1); g_fire(3, 3)
        g_wait(2, 2); s_fire(2, 2); s_wait(0, 0); g_fire(4, 0)
        g_wait(3, 3); s_fire(3, 3); s_wait(1, 1); g_fire(5, 1)

        def mstep(i4, carry2):
            b = i4 * NB
            for j in range(NB):
                ci = b + j
                jd = (j + 2) % NB
                g_wait(ci, j)
                s_fire(ci, j)
                s_wait(ci - 2, jd)
                g_fire(ci + 2, jd)
            return carry2

        lax.fori_loop(1, SUP // NB - 1, mstep, 0)

        base = SUP - NB
        g_wait(base, 0); s_fire(base, 0); s_wait(base - 2, 2)
        g_fire(base + 2, 2)
        g_wait(base + 1, 1); s_fire(base + 1, 1); s_wait(base - 1, 3)
        g_fire(base + 3, 3)
        g_wait(base + 2, 2); s_fire(base + 2, 2); s_wait(base, 0)
        g_wait(base + 3, 3); s_fire(base + 3, 3); s_wait(base + 1, 1)
        s_wait(base + 2, 2)
        s_wait(base + 3, 3)
        return carry

    lax.fori_loop(0, NSUP, sup_main, 0)
    plsc.subcore_barrier()
    write_acc(sums_out)

    if with_counts:
        zero_acc()
        pltpu.sync_copy(ones_h, r0)

        def sup_cnt(t, carry):
            pltpu.sync_copy(dst_c.at[wid, t], dst_idx)
            descs = [
                pltpu.async_copy(r0, acc.at[dst_idx.at[j, 0]], s0, add=True)
                for j in range(SUP)
            ]
            for d in descs:
                d.wait()
            return carry

        lax.fori_loop(0, NSUP, sup_cnt, 0)
        plsc.subcore_barrier()
        write_acc(cnt_out)


@functools.cache
def _seg_kernels():
    mesh = plsc.VectorSubcoreMesh(core_axis_name="c", subcore_axis_name="s",
                                  num_cores=NC, num_subcores=NS)
    base_scratch = (
        pltpu.VMEM((SUP, 1, CH), jnp.int32),
        pltpu.VMEM((SUP, 1, CH), jnp.int32),
        pltpu.VMEM((CH, D), jnp.float32),
        pltpu.VMEM((CH, D), jnp.float32),
        pltpu.VMEM((CH, D), jnp.float32),
        pltpu.VMEM((CH, D), jnp.float32),
        pltpu.VMEM_SHARED((NP, D), jnp.float32),
    ) + (pltpu.SemaphoreType.DMA,) * 8
    with_counts = pl.kernel(
        functools.partial(_seg_body, True),
        out_type=(jax.ShapeDtypeStruct((2 * NP, D), jnp.float32),
                  jax.ShapeDtypeStruct((2 * NP, D), jnp.float32)),
        mesh=mesh,
        scratch_types=base_scratch,
    )
    plain = pl.kernel(
        functools.partial(_seg_body, False),
        out_type=jax.ShapeDtypeStruct((2 * NP, D), jnp.float32),
        mesh=mesh,
        scratch_types=base_scratch,
    )
    return with_counts, plain

BLK = 1000


def _inv_counts(cnt_blk):
    return 1.0 / jnp.clip(cnt_blk[:, 0:1], 1.0, None)


def _tc1_body(sums, cnt, x, w1p, w1n, b1p, b1n, z_out):
    f32 = jnp.float32
    ap = sums[0] * _inv_counts(cnt[0])
    an = sums[1] * _inv_counts(cnt[1])
    xb = x[...]
    outp = (jnp.dot(ap, w1p[0:D, :], preferred_element_type=f32)
            + jnp.dot(xb, w1p[D:2 * D, :], preferred_element_type=f32)
            + b1p[...])
    outn = (jnp.dot(an, w1n[0:D, :], preferred_element_type=f32)
            + jnp.dot(xb, w1n[D:2 * D, :], preferred_element_type=f32)
            + b1n[...])
    z_out[...] = jnp.maximum(jnp.concatenate([outp, outn], axis=1), 0.0)


def _tc2_body(sums, cnt, z, w2p, w2n, b2p, b2n, out):
    f32 = jnp.float32
    mp = sums[0] * _inv_counts(cnt[0])
    mn = sums[1] * _inv_counts(cnt[1])
    zb = z[...]
    outp = (jnp.dot(mp[:, :H2], w2p[0:H2, :], preferred_element_type=f32)
            + jnp.dot(mn[:, H2:], w2p[H2:2 * H2, :], preferred_element_type=f32)
            + jnp.dot(zb[:, :H2], w2p[2 * H2:, :], preferred_element_type=f32)
            + b2p[...])
    outn = (jnp.dot(mp[:, H2:], w2n[0:H2, :], preferred_element_type=f32)
            + jnp.dot(mn[:, :H2], w2n[H2:2 * H2, :], preferred_element_type=f32)
            + jnp.dot(zb[:, H2:], w2n[2 * H2:, :], preferred_element_type=f32)
            + b2n[...])
    out[...] = jnp.maximum(jnp.concatenate([outp, outn], axis=1), 0.0)


def _row_spec(w):
    return pl.BlockSpec((BLK, w), lambda i: (i, 0))


def _pair_spec(w):
    return pl.BlockSpec((2, BLK, w), lambda i: (0, i, 0))


def _full_spec(r, c):
    return pl.BlockSpec((r, c), lambda i: (0, 0))


def _make_tc(body, wrows):
    return pl.pallas_call(
        body,
        grid=(N // BLK,),
        in_specs=[
            _pair_spec(D), _pair_spec(D), _row_spec(D),
            _full_spec(wrows, H2), _full_spec(wrows, H2),
            _full_spec(1, H2), _full_spec(1, H2),
        ],
        out_specs=_row_spec(D),
        out_shape=jax.ShapeDtypeStruct((N, D), jnp.float32),
    )


_tc1 = _make_tc(_tc1_body, 2 * D)
_tc2 = _make_tc(_tc2_body, 3 * H2)


def kernel(x, pos_edge_index, neg_edge_index, W1p, b1p, W1n, b1n,
           W2p, b2p, W2n, b2n):
    src = jnp.concatenate([pos_edge_index[0], neg_edge_index[0]]).reshape(
        NC * NS, NSUP, SUP, 1, CH)
    dst = jnp.concatenate([pos_edge_index[1], neg_edge_index[1]]).reshape(
        NC * NS, NSUP, SUP, 1, CH)

    seg_counts, seg_plain = _seg_kernels()
    zrows = jnp.zeros((RPW, D), jnp.float32)
    ones_h = jnp.ones((CH, D), jnp.float32)
    sums1, cnts = seg_counts(x, src, dst, zrows, ones_h)
    sums1 = sums1.reshape(2, NP, D)
    cnts = cnts.reshape(2, NP, D)
    z = _tc1(sums1, cnts, x, W1p, W1n,
             b1p.reshape(1, H2), b1n.reshape(1, H2))
    sums2 = seg_plain(z, src, dst, zrows).reshape(2, NP, D)
    out = _tc2(sums2, cnts, z, W2p, W2n,
               b2p.reshape(1, H2), b2n.reshape(1, H2))
    return out


# speedup vs baseline: 1.2517x; 1.2517x over previous
import functools
import jax
import jax.numpy as jnp
from jax import lax
from jax.experimental import pallas as pl
from jax.experimental.pallas import tpu as pltpu
from jax.experimental.pallas import tpu_sc as plsc

N = 10000
D = 128
H2 = 64
E = 320000
NC, NS = 2, 16
NP = 10240
EPW = (2 * E) // (NC * NS)
CH = 80
NCHUNK = EPW // CH
RPW = NP // NS
SUP = 50
NSUP = NCHUNK // SUP


def _seg_body(with_counts, *refs):
    if with_counts:
        (feat, src_c, dst_c, zrows, ones_h, sums_out, cnt_out,
         src_idx, dst_idx, rows0, rows1, acc, sem0, sem1) = refs
    else:
        (feat, src_c, dst_c, zrows, sums_out,
         src_idx, dst_idx, rows0, rows1, acc, sem0, sem1) = refs
    c = lax.axis_index("c")
    s = lax.axis_index("s")
    wid = c * NS + s
    row0 = c * NP + s * RPW

    def zero_acc():
        pltpu.sync_copy(zrows, acc.at[pl.ds(s * RPW, RPW)])
        plsc.subcore_barrier()

    def write_acc(out):
        pltpu.sync_copy(acc.at[pl.ds(s * RPW, RPW)], out.at[pl.ds(row0, RPW)])
        plsc.subcore_barrier()

    zero_acc()

    def sup_main(t, carry):
        pltpu.sync_copy(src_c.at[wid, t], src_idx)
        pltpu.sync_copy(dst_c.at[wid, t], dst_idx)
        pltpu.async_copy(feat.at[src_idx.at[0, 0]], rows0, sem0)
        pltpu.async_copy(feat.at[src_idx.at[1, 0]], rows1, sem1)

        def mstep(i2, carry2):
            c0 = 2 * i2
            pltpu.make_async_copy(
                feat.at[src_idx.at[c0, 0]], rows0, sem0).wait()
            pltpu.sync_copy(rows0, acc.at[dst_idx.at[c0, 0]], add=True)

            @pl.when(i2 < SUP // 2 - 1)
            def _():
                pltpu.async_copy(feat.at[src_idx.at[c0 + 2, 0]], rows0, sem0)

            pltpu.make_async_copy(
                feat.at[src_idx.at[c0 + 1, 0]], rows1, sem1).wait()
            pltpu.sync_copy(rows1, acc.at[dst_idx.at[c0 + 1, 0]], add=True)

            @pl.when(i2 < SUP // 2 - 1)
            def _():
                pltpu.async_copy(feat.at[src_idx.at[c0 + 3, 0]], rows1, sem1)

            return carry2

        lax.fori_loop(0, SUP // 2, mstep, 0)
        return carry

    lax.fori_loop(0, NSUP, sup_main, 0)
    plsc.subcore_barrier()
    write_acc(sums_out)

    if with_counts:
        zero_acc()
        pltpu.sync_copy(ones_h, rows0)

        def sup_cnt(t, carry):
            pltpu.sync_copy(dst_c.at[wid, t], dst_idx)
            descs = [
                pltpu.async_copy(rows0, acc.at[dst_idx.at[j, 0]],
                                 sem0, add=True)
                for j in range(SUP)
            ]
            for d in descs:
                d.wait()
            return carry

        lax.fori_loop(0, NSUP, sup_cnt, 0)
        plsc.subcore_barrier()
        write_acc(cnt_out)


@functools.cache
def _seg_kernels():
    mesh = plsc.VectorSubcoreMesh(core_axis_name="c", subcore_axis_name="s",
                                  num_cores=NC, num_subcores=NS)
    base_scratch = (
        pltpu.VMEM((SUP, 1, CH), jnp.int32),
        pltpu.VMEM((SUP, 1, CH), jnp.int32),
        pltpu.VMEM((CH, D), jnp.float32),
        pltpu.VMEM((CH, D), jnp.float32),
        pltpu.VMEM_SHARED((NP, D), jnp.float32),
        pltpu.SemaphoreType.DMA,
        pltpu.SemaphoreType.DMA,
    )
    with_counts = pl.kernel(
        functools.partial(_seg_body, True),
        out_type=(jax.ShapeDtypeStruct((2 * NP, D), jnp.float32),
                  jax.ShapeDtypeStruct((2 * NP, D), jnp.float32)),
        mesh=mesh,
        scratch_types=base_scratch,
    )
    plain = pl.kernel(
        functools.partial(_seg_body, False),
        out_type=jax.ShapeDtypeStruct((2 * NP, D), jnp.float32),
        mesh=mesh,
        scratch_types=base_scratch,
    )
    return with_counts, plain

BLK = 1000


def _inv_counts(cnt_blk):
    return 1.0 / jnp.clip(cnt_blk[:, 0:1], 1.0, None)


def _tc1_body(sums, cnt, x, w1p, w1n, b1p, b1n, z_out):
    f32 = jnp.float32
    ap = sums[0] * _inv_counts(cnt[0])
    an = sums[1] * _inv_counts(cnt[1])
    xb = x[...]
    outp = (jnp.dot(ap, w1p[0:D, :], preferred_element_type=f32)
            + jnp.dot(xb, w1p[D:2 * D, :], preferred_element_type=f32)
            + b1p[...])
    outn = (jnp.dot(an, w1n[0:D, :], preferred_element_type=f32)
            + jnp.dot(xb, w1n[D:2 * D, :], preferred_element_type=f32)
            + b1n[...])
    z_out[...] = jnp.maximum(jnp.concatenate([outp, outn], axis=1), 0.0)


def _tc2_body(sums, cnt, z, w2p, w2n, b2p, b2n, out):
    f32 = jnp.float32
    mp = sums[0] * _inv_counts(cnt[0])
    mn = sums[1] * _inv_counts(cnt[1])
    zb = z[...]
    outp = (jnp.dot(mp[:, :H2], w2p[0:H2, :], preferred_element_type=f32)
            + jnp.dot(mn[:, H2:], w2p[H2:2 * H2, :], preferred_element_type=f32)
            + jnp.dot(zb[:, :H2], w2p[2 * H2:, :], preferred_element_type=f32)
            + b2p[...])
    outn = (jnp.dot(mp[:, H2:], w2n[0:H2, :], preferred_element_type=f32)
            + jnp.dot(mn[:, :H2], w2n[H2:2 * H2, :], preferred_element_type=f32)
            + jnp.dot(zb[:, H2:], w2n[2 * H2:, :], preferred_element_type=f32)
            + b2n[...])
    out[...] = jnp.maximum(jnp.concatenate([outp, outn], axis=1), 0.0)


def _row_spec(w):
    return pl.BlockSpec((BLK, w), lambda i: (i, 0))


def _pair_spec(w):
    return pl.BlockSpec((2, BLK, w), lambda i: (0, i, 0))


def _full_spec(r, c):
    return pl.BlockSpec((r, c), lambda i: (0, 0))


def _make_tc(body, wrows):
    return pl.pallas_call(
        body,
        grid=(N // BLK,),
        in_specs=[
            _pair_spec(D), _pair_spec(D), _row_spec(D),
            _full_spec(wrows, H2), _full_spec(wrows, H2),
            _full_spec(1, H2), _full_spec(1, H2),
        ],
        out_specs=_row_spec(D),
        out_shape=jax.ShapeDtypeStruct((N, D), jnp.float32),
    )


_tc1 = _make_tc(_tc1_body, 2 * D)
_tc2 = _make_tc(_tc2_body, 3 * H2)


def kernel(x, pos_edge_index, neg_edge_index, W1p, b1p, W1n, b1n,
           W2p, b2p, W2n, b2n):
    src = jnp.concatenate([pos_edge_index[0], neg_edge_index[0]]).reshape(
        NC * NS, NSUP, SUP, 1, CH)
    dst = jnp.concatenate([pos_edge_index[1], neg_edge_index[1]]).reshape(
        NC * NS, NSUP, SUP, 1, CH)

    seg_counts, seg_plain = _seg_kernels()
    zrows = jnp.zeros((RPW, D), jnp.float32)
    ones_h = jnp.ones((CH, D), jnp.float32)
    sums1, cnts = seg_counts(x, src, dst, zrows, ones_h)
    sums1 = sums1.reshape(2, NP, D)
    cnts = cnts.reshape(2, NP, D)
    z = _tc1(sums1, cnts, x, W1p, W1n,
             b1p.reshape(1, H2), b1n.reshape(1, H2))
    sums2 = seg_plain(z, src, dst, zrows).reshape(2, NP, D)
    out = _tc2(sums2, cnts, z, W2p, W2n,
               b2p.reshape(1, H2), b2n.reshape(1, H2))
    return out


# speedup vs baseline: 1.4091x; 1.1257x over previous
import functools
import jax
import jax.numpy as jnp
from jax import lax
from jax.experimental import pallas as pl
from jax.experimental.pallas import tpu as pltpu
from jax.experimental.pallas import tpu_sc as plsc

N = 10000
D = 128
H2 = 64
E = 320000
NC, NS = 2, 16
NP = 10240
CH = 128
EPW = 20480
EPAD = NS * EPW - E
NCHUNK = EPW // CH
RPW = NP // NS
SUP = 40
NSUP = NCHUNK // SUP


def _seg_body(with_counts, *refs):
    if with_counts:
        (feat, src_c, dst_c, zrows, ones_h, sums_out, cnt_out,
         src_idx, dst_idx, rows0, rows1, acc, sem0, sem1) = refs
    else:
        (feat, src_c, dst_c, zrows, sums_out,
         src_idx, dst_idx, rows0, rows1, acc, sem0, sem1) = refs
    c = lax.axis_index("c")
    s = lax.axis_index("s")
    wid = c * NS + s
    row0 = c * NP + s * RPW

    def zero_acc():
        pltpu.sync_copy(zrows, acc.at[pl.ds(s * RPW, RPW)])
        plsc.subcore_barrier()

    def write_acc(out):
        pltpu.sync_copy(acc.at[pl.ds(s * RPW, RPW)], out.at[pl.ds(row0, RPW)])
        plsc.subcore_barrier()

    zero_acc()

    def sup_main(t, carry):
        pltpu.sync_copy(src_c.at[wid, t], src_idx)
        pltpu.sync_copy(dst_c.at[wid, t], dst_idx)
        pltpu.async_copy(feat.at[src_idx.at[0, 0]], rows0, sem0)
        pltpu.async_copy(feat.at[src_idx.at[1, 0]], rows1, sem1)

        def mstep(i2, carry2):
            c0 = 2 * i2
            pltpu.make_async_copy(
                feat.at[src_idx.at[c0, 0]], rows0, sem0).wait()
            pltpu.sync_copy(rows0, acc.at[dst_idx.at[c0, 0]], add=True)

            @pl.when(i2 < SUP // 2 - 1)
            def _():
                pltpu.async_copy(feat.at[src_idx.at[c0 + 2, 0]], rows0, sem0)

            pltpu.make_async_copy(
                feat.at[src_idx.at[c0 + 1, 0]], rows1, sem1).wait()
            pltpu.sync_copy(rows1, acc.at[dst_idx.at[c0 + 1, 0]], add=True)

            @pl.when(i2 < SUP // 2 - 1)
            def _():
                pltpu.async_copy(feat.at[src_idx.at[c0 + 3, 0]], rows1, sem1)

            return carry2

        lax.fori_loop(0, SUP // 2, mstep, 0)
        return carry

    lax.fori_loop(0, NSUP, sup_main, 0)
    plsc.subcore_barrier()
    write_acc(sums_out)

    if with_counts:
        zero_acc()
        pltpu.sync_copy(ones_h, rows0)

        def sup_cnt(t, carry):
            pltpu.sync_copy(dst_c.at[wid, t], dst_idx)
            descs = [
                pltpu.async_copy(rows0, acc.at[dst_idx.at[j, 0]],
                                 sem0, add=True)
                for j in range(SUP)
            ]
            for d in descs:
                d.wait()
            return carry

        lax.fori_loop(0, NSUP, sup_cnt, 0)
        plsc.subcore_barrier()
        write_acc(cnt_out)


@functools.cache
def _seg_kernels():
    mesh = plsc.VectorSubcoreMesh(core_axis_name="c", subcore_axis_name="s",
                                  num_cores=NC, num_subcores=NS)
    base_scratch = (
        pltpu.VMEM((SUP, 1, CH), jnp.int32),
        pltpu.VMEM((SUP, 1, CH), jnp.int32),
        pltpu.VMEM((CH, D), jnp.float32),
        pltpu.VMEM((CH, D), jnp.float32),
        pltpu.VMEM_SHARED((NP, D), jnp.float32),
        pltpu.SemaphoreType.DMA,
        pltpu.SemaphoreType.DMA,
    )
    with_counts = pl.kernel(
        functools.partial(_seg_body, True),
        out_type=(jax.ShapeDtypeStruct((2 * NP, D), jnp.float32),
                  jax.ShapeDtypeStruct((2 * NP, D), jnp.float32)),
        mesh=mesh,
        scratch_types=base_scratch,
    )
    plain = pl.kernel(
        functools.partial(_seg_body, False),
        out_type=jax.ShapeDtypeStruct((2 * NP, D), jnp.float32),
        mesh=mesh,
        scratch_types=base_scratch,
    )
    return with_counts, plain

BLK = 1024


def _inv_counts(cnt_blk):
    return 1.0 / jnp.clip(cnt_blk[:, 0:1], 1.0, None)


def _tc1_body(sums, cnt, x, w1p, w1n, b1p, b1n, z_out):
    f32 = jnp.float32
    ap = sums[0] * _inv_counts(cnt[0])
    an = sums[1] * _inv_counts(cnt[1])
    xb = x[...]
    outp = (jnp.dot(ap, w1p[0:D, :], preferred_element_type=f32)
            + jnp.dot(xb, w1p[D:2 * D, :], preferred_element_type=f32)
            + b1p[...])
    outn = (jnp.dot(an, w1n[0:D, :], preferred_element_type=f32)
            + jnp.dot(xb, w1n[D:2 * D, :], preferred_element_type=f32)
            + b1n[...])
    z_out[...] = jnp.maximum(jnp.concatenate([outp, outn], axis=1), 0.0)


def _tc2_body(sums, cnt, z, w2p, w2n, b2p, b2n, out):
    f32 = jnp.float32
    mp = sums[0] * _inv_counts(cnt[0])
    mn = sums[1] * _inv_counts(cnt[1])
    zb = z[...]
    outp = (jnp.dot(mp[:, :H2], w2p[0:H2, :], preferred_element_type=f32)
            + jnp.dot(mn[:, H2:], w2p[H2:2 * H2, :], preferred_element_type=f32)
            + jnp.dot(zb[:, :H2], w2p[2 * H2:, :], preferred_element_type=f32)
            + b2p[...])
    outn = (jnp.dot(mp[:, H2:], w2n[0:H2, :], preferred_element_type=f32)
            + jnp.dot(mn[:, :H2], w2n[H2:2 * H2, :], preferred_element_type=f32)
            + jnp.dot(zb[:, H2:], w2n[2 * H2:, :], preferred_element_type=f32)
            + b2n[...])
    out[...] = jnp.maximum(jnp.concatenate([outp, outn], axis=1), 0.0)


def _row_spec(w):
    return pl.BlockSpec((BLK, w), lambda i: (i, 0))


def _pair_spec(w):
    return pl.BlockSpec((2, BLK, w), lambda i: (0, i, 0))


def _full_spec(r, c):
    return pl.BlockSpec((r, c), lambda i: (0, 0))


def _make_tc(body, wrows):
    return pl.pallas_call(
        body,
        grid=(NP // BLK,),
        in_specs=[
            _pair_spec(D), _pair_spec(D), _row_spec(D),
            _full_spec(wrows, H2), _full_spec(wrows, H2),
            _full_spec(1, H2), _full_spec(1, H2),
        ],
        out_specs=_row_spec(D),
        out_shape=jax.ShapeDtypeStruct((NP, D), jnp.float32),
    )


_tc1 = _make_tc(_tc1_body, 2 * D)
_tc2 = _make_tc(_tc2_body, 3 * H2)


def kernel(x, pos_edge_index, neg_edge_index, W1p, b1p, W1n, b1n,
           W2p, b2p, W2n, b2n):
    pad = (jnp.arange(EPAD, dtype=jnp.int32) % (NP - N)) + N

    def edges(row):
        return jnp.concatenate(
            [pos_edge_index[row], pad, neg_edge_index[row], pad]).reshape(
                NC * NS, NSUP, SUP, 1, CH)

    src = edges(0)
    dst = edges(1)
    xp = jnp.pad(x, ((0, NP - N), (0, 0)))

    seg_counts, seg_plain = _seg_kernels()
    zrows = jnp.zeros((RPW, D), jnp.float32)
    ones_h = jnp.ones((CH, D), jnp.float32)
    sums1, cnts = seg_counts(xp, src, dst, zrows, ones_h)
    sums1 = sums1.reshape(2, NP, D)
    cnts = cnts.reshape(2, NP, D)
    z = _tc1(sums1, cnts, xp, W1p, W1n,
             b1p.reshape(1, H2), b1n.reshape(1, H2))
    sums2 = seg_plain(z, src, dst, zrows).reshape(2, NP, D)
    out = _tc2(sums2, cnts, z, W2p, W2n,
               b2p.reshape(1, H2), b2n.reshape(1, H2))
    return out[:N]
